# baseline (speedup 1.0000x reference)
import functools

import jax
import jax.numpy as jnp
from jax import lax
from jax.experimental import pallas as pl
from jax.experimental.pallas import tpu as pltpu
from jax.experimental.pallas import tpu_sc as plsc

N_TOKENS = 2048
D_MODEL = 768
D_FF = 1024
N_EXPERTS = 16
TOP_K = 2

CHUNK = 128
N_FLAT = N_TOKENS * TOP_K
P_MAX = 6144
N_CHUNKS = P_MAX // CHUNK

SC_NC = 2
SC_NS = 16
SC_NW = SC_NC * SC_NS


def _router_kernel(x_ref, wr_ref, w1_ref, w2_ref, a1_ref, a2_ref):
    x = x_ref[...]
    wr = wr_ref[...]
    logits = lax.dot_general(x, wr, (((1,), (1,)), ((), ())),
                             preferred_element_type=jnp.float32)
    m = jnp.max(logits, axis=-1, keepdims=True)
    e = jnp.exp(logits - m)
    p = e / jnp.sum(e, axis=-1, keepdims=True)
    ids = lax.broadcasted_iota(jnp.int32, p.shape, 1)
    a1 = jnp.argmax(p, axis=-1)
    oh1 = ids == a1[:, None]
    w1 = jnp.sum(jnp.where(oh1, p, 0.0), axis=-1)
    a2 = jnp.argmax(jnp.where(oh1, -1.0, p), axis=-1)
    oh2 = ids == a2[:, None]
    w2 = jnp.sum(jnp.where(oh2, p, 0.0), axis=-1)
    s = w1 + w2 + 1e-9
    w1_ref[...] = (w1 / s)[:, None]
    w2_ref[...] = (w2 / s)[:, None]
    a1_ref[...] = a1[:, None]
    a2_ref[...] = a2[:, None]


def _sc_gather_body(x_hbm, ids_hbm, out_hbm, idx_v, rows_v, sem):
    wid = lax.axis_index("s") * SC_NC + lax.axis_index("c")
    b_per_w = P_MAX // SC_NW
    sb = b_per_w // 2
    base = wid * b_per_w
    for j in range(2):
        off = base + j * sb
        pltpu.sync_copy(ids_hbm.at[pl.ds(off, sb)], idx_v)
        pltpu.async_copy(x_hbm.at[idx_v], rows_v, sem).wait()
        pltpu.sync_copy(rows_v, out_hbm.at[pl.ds(off, sb)])


def _sc_gather(x2, row_token):
    mesh = plsc.VectorSubcoreMesh(core_axis_name="c", subcore_axis_name="s")
    sb = P_MAX // SC_NW // 2
    return pl.kernel(
        _sc_gather_body,
        out_type=jax.ShapeDtypeStruct((P_MAX, D_MODEL), jnp.float32),
        mesh=mesh,
        scratch_types=[
            pltpu.VMEM((sb,), jnp.int32),
            pltpu.VMEM((sb, D_MODEL), jnp.float32),
            pltpu.SemaphoreType.DMA,
        ],
    )(x2, row_token)


def _ffn_kernel(ce_ref, nreal_ref, xs_ref, w_ref, w1_ref, b1_ref, w2_ref,
                b2_ref, y_ref):
    c = pl.program_id(0)

    @pl.when(c < nreal_ref[0])
    def _():
        xg = xs_ref[...]
        h = lax.dot_general(xg, w1_ref[0], (((1,), (0,)), ((), ())),
                            preferred_element_type=jnp.float32)
        h = jnp.maximum(h + b1_ref[0], 0.0)
        y = lax.dot_general(h, w2_ref[0], (((1,), (0,)), ((), ())),
                            preferred_element_type=jnp.float32)
        y_ref[...] = w_ref[...] * (y + b2_ref[0])


def _sc_combine_body(y_hbm, p0_hbm, p1_hbm, out_hbm, i0_v, i1_v, r0_v, r1_v,
                     sem0, sem1):
    wid = lax.axis_index("s") * SC_NC + lax.axis_index("c")
    t_per_w = N_TOKENS // SC_NW
    base = wid * t_per_w
    pltpu.sync_copy(p0_hbm.at[pl.ds(base, t_per_w)], i0_v)
    pltpu.sync_copy(p1_hbm.at[pl.ds(base, t_per_w)], i1_v)
    d0 = pltpu.async_copy(y_hbm.at[i0_v], r0_v, sem0)
    d1 = pltpu.async_copy(y_hbm.at[i1_v], r1_v, sem1)
    d0.wait()
    d1.wait()
    ncol = D_MODEL // 16

    def row_body(r, _):
        for cb in range(ncol):
            sl = pl.ds(cb * 16, 16)
            r0_v[r, sl] = r0_v[r, sl] + r1_v[r, sl]
        return 0

    lax.fori_loop(0, t_per_w, row_body, 0)
    pltpu.sync_copy(r0_v, out_hbm.at[pl.ds(base, t_per_w)])


def _sc_combine(y_rows, pos0, pos1):
    mesh = plsc.VectorSubcoreMesh(core_axis_name="c", subcore_axis_name="s")
    t_per_w = N_TOKENS // SC_NW
    return pl.kernel(
        _sc_combine_body,
        out_type=jax.ShapeDtypeStruct((N_TOKENS, D_MODEL), jnp.float32),
        mesh=mesh,
        scratch_types=[
            pltpu.VMEM((t_per_w,), jnp.int32),
            pltpu.VMEM((t_per_w,), jnp.int32),
            pltpu.VMEM((t_per_w, D_MODEL), jnp.float32),
            pltpu.VMEM((t_per_w, D_MODEL), jnp.float32),
            pltpu.SemaphoreType.DMA,
            pltpu.SemaphoreType.DMA,
        ],
    )(y_rows, pos0, pos1)


@jax.jit
def kernel(x, Wr, W1, b1, W2, b2):
    B, N, C = x.shape
    E, _, F = W1.shape
    x2 = x.reshape(N, C)

    w1n, w2n, a1, a2 = pl.pallas_call(
        _router_kernel,
        out_shape=(
            jax.ShapeDtypeStruct((N, 1), jnp.float32),
            jax.ShapeDtypeStruct((N, 1), jnp.float32),
            jax.ShapeDtypeStruct((N, 1), jnp.int32),
            jax.ShapeDtypeStruct((N, 1), jnp.int32),
        ),
    )(x2, Wr)

    a = jnp.concatenate([a1, a2], axis=1).reshape(N_FLAT)
    wf = jnp.concatenate([w1n, w2n], axis=1).reshape(N_FLAT)
    oh = (a[:, None] == jnp.arange(E, dtype=jnp.int32)[None, :])
    cum = jnp.cumsum(oh.astype(jnp.int32), axis=0)
    rank = jnp.sum(jnp.where(oh, cum - 1, 0), axis=1)
    sizes = cum[-1]
    psizes = ((sizes + CHUNK - 1) // CHUNK) * CHUNK
    pends = jnp.cumsum(psizes)
    poff = pends - psizes
    pos = poff[a] + rank
    row_token = jnp.zeros((P_MAX,), jnp.int32).at[pos].set(
        jnp.arange(N_FLAT, dtype=jnp.int32) // TOP_K)
    row_weight = jnp.zeros((P_MAX, 1), jnp.float32).at[pos, 0].set(wf)
    n_real = (pends[-1] // CHUNK).astype(jnp.int32)
    chunk_ends = pends // CHUNK
    chunk_expert = jnp.minimum(
        jnp.sum(chunk_ends[None, :] <= jnp.arange(N_CHUNKS)[:, None],
                axis=1), E - 1).astype(jnp.int32)
    pos2 = pos.reshape(N, TOP_K)
    pos_a = pos2[:, 0]
    pos_b = pos2[:, 1]

    x_rows = _sc_gather(x2, row_token)

    y_rows = pl.pallas_call(
        _ffn_kernel,
        grid_spec=pltpu.PrefetchScalarGridSpec(
            num_scalar_prefetch=2,
            grid=(N_CHUNKS,),
            in_specs=[
                pl.BlockSpec((CHUNK, C), lambda c, ce, nr: (c, 0)),
                pl.BlockSpec((CHUNK, 1), lambda c, ce, nr: (c, 0)),
                pl.BlockSpec((1, C, F), lambda c, ce, nr: (ce[c], 0, 0)),
                pl.BlockSpec((1, 1, F), lambda c, ce, nr: (ce[c], 0, 0)),
                pl.BlockSpec((1, F, C), lambda c, ce, nr: (ce[c], 0, 0)),
                pl.BlockSpec((1, 1, C), lambda c, ce, nr: (ce[c], 0, 0)),
            ],
            out_specs=pl.BlockSpec((CHUNK, C), lambda c, ce, nr: (c, 0)),
        ),
        out_shape=jax.ShapeDtypeStruct((P_MAX, C), jnp.float32),
        compiler_params=pltpu.CompilerParams(
            dimension_semantics=("arbitrary",),
        ),
    )(chunk_expert, n_real.reshape(1), x_rows, row_weight, W1,
      b1.reshape(E, 1, F), W2, b2.reshape(E, 1, C))

    out = _sc_combine(y_rows, pos_a, pos_b)
    return out.reshape(B, N, C)


# speedup vs baseline: 1.2145x; 1.2145x over previous
import functools

import jax
import jax.numpy as jnp
from jax import lax
from jax.experimental import pallas as pl
from jax.experimental.pallas import tpu as pltpu
from jax.experimental.pallas import tpu_sc as plsc

N_TOKENS = 2048
D_MODEL = 768
D_FF = 1024
N_EXPERTS = 16
TOP_K = 2

CHUNK = 128
N_FLAT = N_TOKENS * TOP_K
P_MAX = 6144
N_CHUNKS = P_MAX // CHUNK

SC_NC = 2
SC_NS = 16
SC_NW = SC_NC * SC_NS


def _router_kernel(x_ref, wr_ref, w1_ref, w2_ref, a1_ref, a2_ref):
    x = x_ref[...]
    wr = wr_ref[...]
    logits = lax.dot_general(x, wr, (((1,), (1,)), ((), ())),
                             preferred_element_type=jnp.float32)
    m = jnp.max(logits, axis=-1, keepdims=True)
    e = jnp.exp(logits - m)
    p = e / jnp.sum(e, axis=-1, keepdims=True)
    ids = lax.broadcasted_iota(jnp.int32, p.shape, 1)
    a1 = jnp.argmax(p, axis=-1)
    oh1 = ids == a1[:, None]
    w1 = jnp.sum(jnp.where(oh1, p, 0.0), axis=-1)
    a2 = jnp.argmax(jnp.where(oh1, -1.0, p), axis=-1)
    oh2 = ids == a2[:, None]
    w2 = jnp.sum(jnp.where(oh2, p, 0.0), axis=-1)
    s = w1 + w2 + 1e-9
    w1_ref[...] = (w1 / s)[:, None]
    w2_ref[...] = (w2 / s)[:, None]
    a1_ref[...] = a1[:, None]
    a2_ref[...] = a2[:, None]


def _sc_gather_body(x_hbm, ids_hbm, out_hbm, idx_v, rows_v, sem):
    wid = lax.axis_index("s") * SC_NC + lax.axis_index("c")
    b_per_w = P_MAX // SC_NW
    sb = b_per_w // 2
    base = wid * b_per_w
    for j in range(2):
        off = base + j * sb
        pltpu.sync_copy(ids_hbm.at[pl.ds(off, sb)], idx_v)
        pltpu.async_copy(x_hbm.at[idx_v], rows_v, sem).wait()
        pltpu.sync_copy(rows_v, out_hbm.at[pl.ds(off, sb)])


def _sc_gather(x2, row_token):
    mesh = plsc.VectorSubcoreMesh(core_axis_name="c", subcore_axis_name="s")
    sb = P_MAX // SC_NW // 2
    return pl.kernel(
        _sc_gather_body,
        out_type=jax.ShapeDtypeStruct((P_MAX, D_MODEL), jnp.float32),
        mesh=mesh,
        scratch_types=[
            pltpu.VMEM((sb,), jnp.int32),
            pltpu.VMEM((sb, D_MODEL), jnp.float32),
            pltpu.SemaphoreType.DMA,
        ],
    )(x2, row_token)


def _ffn_kernel(ce_ref, nreal_ref, xs_ref, w_ref, w1_ref, b1_ref, w2_ref,
                b2_ref, y_ref):
    c = pl.program_id(0)

    @pl.when(c < nreal_ref[0])
    def _():
        xg = xs_ref[...]
        h = lax.dot_general(xg, w1_ref[0], (((1,), (0,)), ((), ())),
                            preferred_element_type=jnp.float32)
        h = jnp.maximum(h + b1_ref[0], 0.0)
        y = lax.dot_general(h, w2_ref[0], (((1,), (0,)), ((), ())),
                            preferred_element_type=jnp.float32)
        y_ref[...] = w_ref[...] * (y + b2_ref[0])


def _sc_combine_body(y_hbm, p0_hbm, p1_hbm, out_hbm, i0_v, i1_v, r0_v, r1_v,
                     sem0, sem1):
    wid = lax.axis_index("s") * SC_NC + lax.axis_index("c")
    t_per_w = N_TOKENS // SC_NW
    base = wid * t_per_w
    pltpu.sync_copy(p0_hbm.at[pl.ds(base, t_per_w)], i0_v)
    pltpu.sync_copy(p1_hbm.at[pl.ds(base, t_per_w)], i1_v)
    d0 = pltpu.async_copy(y_hbm.at[i0_v], r0_v, sem0)
    d1 = pltpu.async_copy(y_hbm.at[i1_v], r1_v, sem1)
    d0.wait()
    d1.wait()
    ncol = D_MODEL // 16

    def row_body(r, _):
        for cb in range(ncol):
            sl = pl.ds(cb * 16, 16)
            r0_v[r, sl] = r0_v[r, sl] + r1_v[r, sl]
        return 0

    lax.fori_loop(0, t_per_w, row_body, 0)
    pltpu.sync_copy(r0_v, out_hbm.at[pl.ds(base, t_per_w)])


def _sc_combine(y_rows, pos0, pos1):
    mesh = plsc.VectorSubcoreMesh(core_axis_name="c", subcore_axis_name="s")
    t_per_w = N_TOKENS // SC_NW
    return pl.kernel(
        _sc_combine_body,
        out_type=jax.ShapeDtypeStruct((N_TOKENS, D_MODEL), jnp.float32),
        mesh=mesh,
        scratch_types=[
            pltpu.VMEM((t_per_w,), jnp.int32),
            pltpu.VMEM((t_per_w,), jnp.int32),
            pltpu.VMEM((t_per_w, D_MODEL), jnp.float32),
            pltpu.VMEM((t_per_w, D_MODEL), jnp.float32),
            pltpu.SemaphoreType.DMA,
            pltpu.SemaphoreType.DMA,
        ],
    )(y_rows, pos0, pos1)


@jax.jit
def kernel(x, Wr, W1, b1, W2, b2):
    B, N, C = x.shape
    E, _, F = W1.shape
    x2 = x.reshape(N, C)

    w1n, w2n, a1, a2 = pl.pallas_call(
        _router_kernel,
        out_shape=(
            jax.ShapeDtypeStruct((N, 1), jnp.float32),
            jax.ShapeDtypeStruct((N, 1), jnp.float32),
            jax.ShapeDtypeStruct((N, 1), jnp.int32),
            jax.ShapeDtypeStruct((N, 1), jnp.int32),
        ),
    )(x2, Wr)

    a = jnp.concatenate([a1, a2], axis=1).reshape(N_FLAT)
    wf = jnp.concatenate([w1n, w2n], axis=1).reshape(N_FLAT)
    oh = (a[:, None] == jnp.arange(E, dtype=jnp.int32)[None, :])
    cum = jnp.cumsum(oh.astype(jnp.int32), axis=0)
    rank = jnp.sum(jnp.where(oh, cum - 1, 0), axis=1)
    sizes = cum[-1]
    psizes = ((sizes + CHUNK - 1) // CHUNK) * CHUNK
    pends = jnp.cumsum(psizes)
    poff = pends - psizes
    pos = poff[a] + rank
    row_token = jnp.zeros((P_MAX,), jnp.int32).at[pos].set(
        jnp.arange(N_FLAT, dtype=jnp.int32) // TOP_K)
    row_weight = jnp.zeros((P_MAX, 1), jnp.float32).at[pos, 0].set(wf)
    n_real = (pends[-1] // CHUNK).astype(jnp.int32)
    chunk_ends = pends // CHUNK
    chunk_expert = jnp.minimum(
        jnp.sum(chunk_ends[None, :] <= jnp.arange(N_CHUNKS)[:, None],
                axis=1), E - 1).astype(jnp.int32)
    pos2 = pos.reshape(N, TOP_K)
    pos_a = pos2[:, 0]
    pos_b = pos2[:, 1]

    x_rows = x2[row_token]

    y_rows = pl.pallas_call(
        _ffn_kernel,
        grid_spec=pltpu.PrefetchScalarGridSpec(
            num_scalar_prefetch=2,
            grid=(N_CHUNKS,),
            in_specs=[
                pl.BlockSpec((CHUNK, C), lambda c, ce, nr: (c, 0)),
                pl.BlockSpec((CHUNK, 1), lambda c, ce, nr: (c, 0)),
                pl.BlockSpec((1, C, F), lambda c, ce, nr: (ce[c], 0, 0)),
                pl.BlockSpec((1, 1, F), lambda c, ce, nr: (ce[c], 0, 0)),
                pl.BlockSpec((1, F, C), lambda c, ce, nr: (ce[c], 0, 0)),
                pl.BlockSpec((1, 1, C), lambda c, ce, nr: (ce[c], 0, 0)),
            ],
            out_specs=pl.BlockSpec((CHUNK, C), lambda c, ce, nr: (c, 0)),
        ),
        out_shape=jax.ShapeDtypeStruct((P_MAX, C), jnp.float32),
        compiler_params=pltpu.CompilerParams(
            dimension_semantics=("arbitrary",),
        ),
    )(chunk_expert, n_real.reshape(1), x_rows, row_weight, W1,
      b1.reshape(E, 1, F), W2, b2.reshape(E, 1, C))

    out = y_rows[pos_a] + y_rows[pos_b]
    return out.reshape(B, N, C)


# speedup vs baseline: 1.7500x; 1.4409x over previous
import functools

import jax
import jax.numpy as jnp
from jax import lax
from jax.experimental import pallas as pl
from jax.experimental.pallas import tpu as pltpu
from jax.experimental.pallas import tpu_sc as plsc

N_TOKENS = 2048
D_MODEL = 768
D_FF = 1024
N_EXPERTS = 16
TOP_K = 2

CHUNK = 128
N_FLAT = N_TOKENS * TOP_K
P_MAX = 6144
N_CHUNKS = P_MAX // CHUNK

SC_NC = 2
SC_NS = 16
SC_NW = SC_NC * SC_NS
T_PER_W = N_TOKENS // SC_NW


def _router_kernel(x_ref, wr_ref, w1_ref, w2_ref, a1_ref, a2_ref):
    x = x_ref[...]
    wr = wr_ref[...]
    logits = lax.dot_general(x, wr, (((1,), (1,)), ((), ())),
                             preferred_element_type=jnp.float32)
    m = jnp.max(logits, axis=-1, keepdims=True)
    e = jnp.exp(logits - m)
    p = e / jnp.sum(e, axis=-1, keepdims=True)
    ids = lax.broadcasted_iota(jnp.int32, p.shape, 1)
    a1 = jnp.argmax(p, axis=-1)
    oh1 = ids == a1[:, None]
    w1 = jnp.sum(jnp.where(oh1, p, 0.0), axis=-1)
    a2 = jnp.argmax(jnp.where(oh1, -1.0, p), axis=-1)
    oh2 = ids == a2[:, None]
    w2 = jnp.sum(jnp.where(oh2, p, 0.0), axis=-1)
    s = w1 + w2 + 1e-9
    w1_ref[...] = (w1 / s)[:, None]
    w2_ref[...] = (w2 / s)[:, None]
    a1_ref[...] = a1[:, None]
    a2_ref[...] = a2[:, None]


def _sc_dispatch_body(x_hbm, pa_hbm, pb_hbm, out_hbm, ia_v, ib_v, rows_v,
                      sema, semb):
    wid = lax.axis_index("s") * SC_NC + lax.axis_index("c")
    base = wid * T_PER_W
    pltpu.sync_copy(pa_hbm.at[pl.ds(base, T_PER_W)], ia_v)
    pltpu.sync_copy(pb_hbm.at[pl.ds(base, T_PER_W)], ib_v)
    pltpu.sync_copy(x_hbm.at[pl.ds(base, T_PER_W)], rows_v)
    da = pltpu.async_copy(rows_v, out_hbm.at[ia_v], sema)
    db = pltpu.async_copy(rows_v, out_hbm.at[ib_v], semb)
    da.wait()
    db.wait()


def _sc_dispatch(x2, pos_a, pos_b):
    mesh = plsc.VectorSubcoreMesh(core_axis_name="c", subcore_axis_name="s")
    return pl.kernel(
        _sc_dispatch_body,
        out_type=jax.ShapeDtypeStruct((P_MAX, D_MODEL), jnp.float32),
        mesh=mesh,
        scratch_types=[
            pltpu.VMEM((T_PER_W,), jnp.int32),
            pltpu.VMEM((T_PER_W,), jnp.int32),
            pltpu.VMEM((T_PER_W, D_MODEL), jnp.float32),
            pltpu.SemaphoreType.DMA,
            pltpu.SemaphoreType.DMA,
        ],
    )(x2, pos_a, pos_b)


def _ffn_kernel(ce_ref, nreal_ref, xs_ref, w1_ref, b1_ref, w2_ref,
                b2_ref, y_ref):
    c = pl.program_id(0)

    @pl.when(c < nreal_ref[0])
    def _():
        xg = xs_ref[...]
        h = lax.dot_general(xg, w1_ref[0], (((1,), (0,)), ((), ())),
                            preferred_element_type=jnp.float32)
        h = jnp.maximum(h + b1_ref[0], 0.0)
        y = lax.dot_general(h, w2_ref[0], (((1,), (0,)), ((), ())),
                            preferred_element_type=jnp.float32)
        y_ref[...] = y + b2_ref[0]


def _sc_combine_body(y_hbm, p0_hbm, p1_hbm, w0_hbm, w1_hbm, out_hbm,
                     i0_v, i1_v, w0_v, w1_v, r0_v, r1_v, sem0, sem1):
    wid = lax.axis_index("s") * SC_NC + lax.axis_index("c")
    base = wid * T_PER_W
    pltpu.sync_copy(p0_hbm.at[pl.ds(base, T_PER_W)], i0_v)
    pltpu.sync_copy(p1_hbm.at[pl.ds(base, T_PER_W)], i1_v)
    pltpu.sync_copy(w0_hbm.at[pl.ds(base, T_PER_W)], w0_v)
    pltpu.sync_copy(w1_hbm.at[pl.ds(base, T_PER_W)], w1_v)
    d0 = pltpu.async_copy(y_hbm.at[i0_v], r0_v, sem0)
    d1 = pltpu.async_copy(y_hbm.at[i1_v], r1_v, sem1)
    d0.wait()
    d1.wait()
    ncol = D_MODEL // 16

    def row_body(r, _):
        w0 = w0_v[r, :]
        w1 = w1_v[r, :]
        for cb in range(ncol):
            sl = pl.ds(cb * 16, 16)
            r0_v[r, sl] = w0 * r0_v[r, sl] + w1 * r1_v[r, sl]
        return 0

    lax.fori_loop(0, T_PER_W, row_body, 0)
    pltpu.sync_copy(r0_v, out_hbm.at[pl.ds(base, T_PER_W)])


def _sc_combine(y_rows, pos0, pos1, w0, w1):
    mesh = plsc.VectorSubcoreMesh(core_axis_name="c", subcore_axis_name="s")
    return pl.kernel(
        _sc_combine_body,
        out_type=jax.ShapeDtypeStruct((N_TOKENS, D_MODEL), jnp.float32),
        mesh=mesh,
        scratch_types=[
            pltpu.VMEM((T_PER_W,), jnp.int32),
            pltpu.VMEM((T_PER_W,), jnp.int32),
            pltpu.VMEM((T_PER_W, 16), jnp.float32),
            pltpu.VMEM((T_PER_W, 16), jnp.float32),
            pltpu.VMEM((T_PER_W, D_MODEL), jnp.float32),
            pltpu.VMEM((T_PER_W, D_MODEL), jnp.float32),
            pltpu.SemaphoreType.DMA,
            pltpu.SemaphoreType.DMA,
        ],
    )(y_rows, pos0, pos1, w0, w1)


@jax.jit
def kernel(x, Wr, W1, b1, W2, b2):
    B, N, C = x.shape
    E, _, F = W1.shape
    x2 = x.reshape(N, C)

    w1n, w2n, a1, a2 = pl.pallas_call(
        _router_kernel,
        out_shape=(
            jax.ShapeDtypeStruct((N, 1), jnp.float32),
            jax.ShapeDtypeStruct((N, 1), jnp.float32),
            jax.ShapeDtypeStruct((N, 1), jnp.int32),
            jax.ShapeDtypeStruct((N, 1), jnp.int32),
        ),
    )(x2, Wr)

    a = jnp.concatenate([a1, a2], axis=1).reshape(N_FLAT)
    oh = (a[:, None] == jnp.arange(E, dtype=jnp.int32)[None, :])
    cum = jnp.cumsum(oh.astype(jnp.int32), axis=0)
    rank = jnp.sum(jnp.where(oh, cum - 1, 0), axis=1)
    sizes = cum[-1]
    psizes = ((sizes + CHUNK - 1) // CHUNK) * CHUNK
    pends = jnp.cumsum(psizes)
    poff = pends - psizes
    pos = (poff[a] + rank).astype(jnp.int32)
    n_real = (pends[-1] // CHUNK).astype(jnp.int32)
    chunk_ends = pends // CHUNK
    chunk_expert = jnp.minimum(
        jnp.sum(chunk_ends[None, :] <= jnp.arange(N_CHUNKS)[:, None],
                axis=1), E - 1).astype(jnp.int32)
    pos2 = pos.reshape(N, TOP_K)
    pos_a = pos2[:, 0]
    pos_b = pos2[:, 1]

    x_rows = _sc_dispatch(x2, pos_a, pos_b)

    y_rows = pl.pallas_call(
        _ffn_kernel,
        grid_spec=pltpu.PrefetchScalarGridSpec(
            num_scalar_prefetch=2,
            grid=(N_CHUNKS,),
            in_specs=[
                pl.BlockSpec((CHUNK, C), lambda c, ce, nr: (c, 0)),
                pl.BlockSpec((1, C, F), lambda c, ce, nr: (ce[c], 0, 0)),
                pl.BlockSpec((1, 1, F), lambda c, ce, nr: (ce[c], 0, 0)),
                pl.BlockSpec((1, F, C), lambda c, ce, nr: (ce[c], 0, 0)),
                pl.BlockSpec((1, 1, C), lambda c, ce, nr: (ce[c], 0, 0)),
            ],
            out_specs=pl.BlockSpec((CHUNK, C), lambda c, ce, nr: (c, 0)),
        ),
        out_shape=jax.ShapeDtypeStruct((P_MAX, C), jnp.float32),
        compiler_params=pltpu.CompilerParams(
            dimension_semantics=("arbitrary",),
        ),
    )(chunk_expert, n_real.reshape(1), x_rows, W1,
      b1.reshape(E, 1, F), W2, b2.reshape(E, 1, C))

    out = _sc_combine(y_rows, pos_a, pos_b,
                      jnp.broadcast_to(w1n, (N, 16)),
                      jnp.broadcast_to(w2n, (N, 16)))
    return out.reshape(B, N, C)


# speedup vs baseline: 1.9042x; 1.0882x over previous
import functools

import jax
import jax.numpy as jnp
from jax import lax
from jax.experimental import pallas as pl
from jax.experimental.pallas import tpu as pltpu
from jax.experimental.pallas import tpu_sc as plsc

N_TOKENS = 2048
D_MODEL = 768
D_FF = 1024
N_EXPERTS = 16
TOP_K = 2

CHUNK = 256
N_FLAT = N_TOKENS * TOP_K
P_MAX = 8192
N_CHUNKS = P_MAX // CHUNK

SC_NC = 2
SC_NS = 16
SC_NW = SC_NC * SC_NS
T_PER_W = N_TOKENS // SC_NW


def _router_kernel(x_ref, wr_ref, w1_ref, w2_ref, a1_ref, a2_ref):
    x = x_ref[...]
    wr = wr_ref[...]
    logits = lax.dot_general(x, wr, (((1,), (1,)), ((), ())),
                             preferred_element_type=jnp.float32)
    m = jnp.max(logits, axis=-1, keepdims=True)
    e = jnp.exp(logits - m)
    p = e / jnp.sum(e, axis=-1, keepdims=True)
    ids = lax.broadcasted_iota(jnp.int32, p.shape, 1)
    a1 = jnp.argmax(p, axis=-1)
    oh1 = ids == a1[:, None]
    w1 = jnp.sum(jnp.where(oh1, p, 0.0), axis=-1)
    a2 = jnp.argmax(jnp.where(oh1, -1.0, p), axis=-1)
    oh2 = ids == a2[:, None]
    w2 = jnp.sum(jnp.where(oh2, p, 0.0), axis=-1)
    s = w1 + w2 + 1e-9
    w1_ref[...] = (w1 / s)[:, None]
    w2_ref[...] = (w2 / s)[:, None]
    a1_ref[...] = a1[:, None]
    a2_ref[...] = a2[:, None]


def _sc_dispatch_body(x_hbm, pa_hbm, pb_hbm, out_hbm, ia_v, ib_v, rows_v,
                      sema, semb):
    wid = lax.axis_index("s") * SC_NC + lax.axis_index("c")
    base = wid * T_PER_W
    pltpu.sync_copy(pa_hbm.at[pl.ds(base, T_PER_W)], ia_v)
    pltpu.sync_copy(pb_hbm.at[pl.ds(base, T_PER_W)], ib_v)
    pltpu.sync_copy(x_hbm.at[pl.ds(base, T_PER_W)], rows_v)
    da = pltpu.async_copy(rows_v, out_hbm.at[ia_v], sema)
    db = pltpu.async_copy(rows_v, out_hbm.at[ib_v], semb)
    da.wait()
    db.wait()


def _sc_dispatch(x2, pos_a, pos_b):
    mesh = plsc.VectorSubcoreMesh(core_axis_name="c", subcore_axis_name="s")
    return pl.kernel(
        _sc_dispatch_body,
        out_type=jax.ShapeDtypeStruct((P_MAX, D_MODEL), jnp.float32),
        mesh=mesh,
        scratch_types=[
            pltpu.VMEM((T_PER_W,), jnp.int32),
            pltpu.VMEM((T_PER_W,), jnp.int32),
            pltpu.VMEM((T_PER_W, D_MODEL), jnp.float32),
            pltpu.SemaphoreType.DMA,
            pltpu.SemaphoreType.DMA,
        ],
    )(x2, pos_a, pos_b)


def _ffn_kernel(ce_ref, nreal_ref, xs_ref, w1_ref, b1_ref, w2_ref,
                b2_ref, y_ref):
    c = pl.program_id(0)

    @pl.when(c < nreal_ref[0])
    def _():
        xg = xs_ref[...]
        h = lax.dot_general(xg, w1_ref[0], (((1,), (0,)), ((), ())),
                            preferred_element_type=jnp.float32)
        h = jnp.maximum(h + b1_ref[0], 0.0)
        y = lax.dot_general(h, w2_ref[0], (((1,), (0,)), ((), ())),
                            preferred_element_type=jnp.float32)
        y_ref[...] = y + b2_ref[0]


def _sc_combine_body(y_hbm, p0_hbm, p1_hbm, w0_hbm, w1_hbm, out_hbm,
                     i0_v, i1_v, w0_v, w1_v, r0_v, r1_v, sem0, sem1):
    wid = lax.axis_index("s") * SC_NC + lax.axis_index("c")
    base = wid * T_PER_W
    pltpu.sync_copy(p0_hbm.at[pl.ds(base, T_PER_W)], i0_v)
    pltpu.sync_copy(p1_hbm.at[pl.ds(base, T_PER_W)], i1_v)
    pltpu.sync_copy(w0_hbm.at[pl.ds(base, T_PER_W)], w0_v)
    pltpu.sync_copy(w1_hbm.at[pl.ds(base, T_PER_W)], w1_v)
    d0 = pltpu.async_copy(y_hbm.at[i0_v], r0_v, sem0)
    d1 = pltpu.async_copy(y_hbm.at[i1_v], r1_v, sem1)
    d0.wait()
    d1.wait()
    ncol = D_MODEL // 16

    def row_body(r, _):
        w0 = w0_v[r, :]
        w1 = w1_v[r, :]
        for cb in range(ncol):
            sl = pl.ds(cb * 16, 16)
            r0_v[r, sl] = w0 * r0_v[r, sl] + w1 * r1_v[r, sl]
        return 0

    lax.fori_loop(0, T_PER_W, row_body, 0)
    pltpu.sync_copy(r0_v, out_hbm.at[pl.ds(base, T_PER_W)])


def _sc_combine(y_rows, pos0, pos1, w0, w1):
    mesh = plsc.VectorSubcoreMesh(core_axis_name="c", subcore_axis_name="s")
    return pl.kernel(
        _sc_combine_body,
        out_type=jax.ShapeDtypeStruct((N_TOKENS, D_MODEL), jnp.float32),
        mesh=mesh,
        scratch_types=[
            pltpu.VMEM((T_PER_W,), jnp.int32),
            pltpu.VMEM((T_PER_W,), jnp.int32),
            pltpu.VMEM((T_PER_W, 16), jnp.float32),
            pltpu.VMEM((T_PER_W, 16), jnp.float32),
            pltpu.VMEM((T_PER_W, D_MODEL), jnp.float32),
            pltpu.VMEM((T_PER_W, D_MODEL), jnp.float32),
            pltpu.SemaphoreType.DMA,
            pltpu.SemaphoreType.DMA,
        ],
    )(y_rows, pos0, pos1, w0, w1)


@jax.jit
def kernel(x, Wr, W1, b1, W2, b2):
    B, N, C = x.shape
    E, _, F = W1.shape
    x2 = x.reshape(N, C)

    w1n, w2n, a1, a2 = pl.pallas_call(
        _router_kernel,
        out_shape=(
            jax.ShapeDtypeStruct((N, 1), jnp.float32),
            jax.ShapeDtypeStruct((N, 1), jnp.float32),
            jax.ShapeDtypeStruct((N, 1), jnp.int32),
            jax.ShapeDtypeStruct((N, 1), jnp.int32),
        ),
    )(x2, Wr)

    a = jnp.concatenate([a1, a2], axis=1).reshape(N_FLAT)
    oh = (a[:, None] == jnp.arange(E, dtype=jnp.int32)[None, :])
    cum = jnp.cumsum(oh.astype(jnp.int32), axis=0)
    rank = jnp.sum(jnp.where(oh, cum - 1, 0), axis=1)
    sizes = cum[-1]
    psizes = ((sizes + CHUNK - 1) // CHUNK) * CHUNK
    pends = jnp.cumsum(psizes)
    poff = pends - psizes
    pos = (poff[a] + rank).astype(jnp.int32)
    n_real = (pends[-1] // CHUNK).astype(jnp.int32)
    chunk_ends = pends // CHUNK
    chunk_expert = jnp.minimum(
        jnp.sum(chunk_ends[None, :] <= jnp.arange(N_CHUNKS)[:, None],
                axis=1), E - 1).astype(jnp.int32)
    pos2 = pos.reshape(N, TOP_K)
    pos_a = pos2[:, 0]
    pos_b = pos2[:, 1]

    x_rows = _sc_dispatch(x2, pos_a, pos_b)

    y_rows = pl.pallas_call(
        _ffn_kernel,
        grid_spec=pltpu.PrefetchScalarGridSpec(
            num_scalar_prefetch=2,
            grid=(N_CHUNKS,),
            in_specs=[
                pl.BlockSpec((CHUNK, C), lambda c, ce, nr: (c, 0)),
                pl.BlockSpec((1, C, F), lambda c, ce, nr: (ce[c], 0, 0)),
                pl.BlockSpec((1, 1, F), lambda c, ce, nr: (ce[c], 0, 0)),
                pl.BlockSpec((1, F, C), lambda c, ce, nr: (ce[c], 0, 0)),
                pl.BlockSpec((1, 1, C), lambda c, ce, nr: (ce[c], 0, 0)),
            ],
            out_specs=pl.BlockSpec((CHUNK, C), lambda c, ce, nr: (c, 0)),
        ),
        out_shape=jax.ShapeDtypeStruct((P_MAX, C), jnp.float32),
        compiler_params=pltpu.CompilerParams(
            dimension_semantics=("arbitrary",),
        ),
    )(chunk_expert, n_real.reshape(1), x_rows, W1,
      b1.reshape(E, 1, F), W2, b2.reshape(E, 1, C))

    out = _sc_combine(y_rows, pos_a, pos_b,
                      jnp.broadcast_to(w1n, (N, 16)),
                      jnp.broadcast_to(w2n, (N, 16)))
    return out.reshape(B, N, C)


# speedup vs baseline: 2.2684x; 1.1912x over previous
import functools

import jax
import jax.numpy as jnp
from jax import lax
from jax.experimental import pallas as pl
from jax.experimental.pallas import tpu as pltpu
from jax.experimental.pallas import tpu_sc as plsc

N_TOKENS = 2048
D_MODEL = 768
D_FF = 1024
N_EXPERTS = 16
TOP_K = 2

CHUNK = 256
N_FLAT = N_TOKENS * TOP_K
P_MAX = 8192
N_CHUNKS = P_MAX // CHUNK

SC_NC = 2
SC_NS = 16
SC_NW = SC_NC * SC_NS
T_PER_W = N_TOKENS // SC_NW


def _cumsum0(v):
    n = v.shape[0]
    sh = 1
    while sh < n:
        shifted = jnp.concatenate(
            [jnp.zeros((sh, v.shape[1]), v.dtype), v[:-sh]], axis=0)
        v = v + shifted
        sh *= 2
    return v


def _router_kernel(x_ref, wr_ref, wa_ref, wb_ref, pa_ref, pb_ref,
                   ce_ref, nr_ref):
    x = x_ref[...]
    wr = wr_ref[...]
    logits = lax.dot_general(x, wr, (((1,), (1,)), ((), ())),
                             preferred_element_type=jnp.float32)
    m = jnp.max(logits, axis=-1, keepdims=True)
    ex = jnp.exp(logits - m)
    p = ex / jnp.sum(ex, axis=-1, keepdims=True)
    ids = lax.broadcasted_iota(jnp.int32, p.shape, 1)
    a1 = jnp.argmax(p, axis=-1)
    oh1 = ids == a1[:, None]
    w1 = jnp.sum(jnp.where(oh1, p, 0.0), axis=-1)
    a2 = jnp.argmax(jnp.where(oh1, -1.0, p), axis=-1)
    oh2 = ids == a2[:, None]
    w2 = jnp.sum(jnp.where(oh2, p, 0.0), axis=-1)
    s = w1 + w2 + 1e-9
    n, e = p.shape
    wa_ref[...] = jnp.broadcast_to((w1 / s)[:, None], (n, 16))
    wb_ref[...] = jnp.broadcast_to((w2 / s)[:, None], (n, 16))

    cnt = oh1.astype(jnp.int32) + oh2.astype(jnp.int32)
    cinc = _cumsum0(cnt)
    cexc = cinc - cnt
    sizes = cinc[n - 1:n, :].astype(jnp.float32)
    chunk_f = jnp.float32(CHUNK)
    psz = jnp.floor((sizes + (chunk_f - 1.0)) / chunk_f) * chunk_f
    rr = lax.broadcasted_iota(jnp.int32, (e, e), 0)
    cc = lax.broadcasted_iota(jnp.int32, (e, e), 1)
    upper = (rr <= cc).astype(jnp.float32)
    pends = lax.dot_general(psz, upper, (((1,), (0,)), ((), ())),
                            preferred_element_type=jnp.float32)
    poff = pends - psz
    rank_a = jnp.sum(jnp.where(oh1, cexc, 0), axis=-1)
    rank_b = jnp.sum(jnp.where(oh2, cexc, 0), axis=-1)
    poff_b = jnp.broadcast_to(poff, (n, e))
    base_a = jnp.sum(jnp.where(oh1, poff_b, 0.0), axis=-1)
    base_b = jnp.sum(jnp.where(oh2, poff_b, 0.0), axis=-1)
    pa_ref[...] = (base_a.astype(jnp.int32) + rank_a)[:, None]
    pb_ref[...] = (base_b.astype(jnp.int32) + rank_b)[:, None]

    cends = (pends / chunk_f).astype(jnp.int32)
    nr_ref[...] = cends[:, e - 1:e]
    ce_cols = lax.broadcasted_iota(jnp.int32, (e, N_CHUNKS), 1)
    cends_t = jnp.broadcast_to(cends.reshape(e, 1), (e, N_CHUNKS))
    ce = jnp.sum((cends_t <= ce_cols).astype(jnp.int32), axis=0,
                 keepdims=True)
    ce_ref[...] = jnp.minimum(ce, e - 1)


def _router(x2, Wr, N, E):
    return pl.pallas_call(
        _router_kernel,
        out_shape=(
            jax.ShapeDtypeStruct((N, 16), jnp.float32),
            jax.ShapeDtypeStruct((N, 16), jnp.float32),
            jax.ShapeDtypeStruct((N, 1), jnp.int32),
            jax.ShapeDtypeStruct((N, 1), jnp.int32),
            jax.ShapeDtypeStruct((1, N_CHUNKS), jnp.int32),
            jax.ShapeDtypeStruct((1, 1), jnp.int32),
        ),
    )(x2, Wr)


def _sc_dispatch_body(x_hbm, pa_hbm, pb_hbm, out_hbm, ia_v, ib_v, rows_v,
                      sema, semb):
    wid = lax.axis_index("s") * SC_NC + lax.axis_index("c")
    base = wid * T_PER_W
    pltpu.sync_copy(pa_hbm.at[pl.ds(base, T_PER_W)], ia_v)
    pltpu.sync_copy(pb_hbm.at[pl.ds(base, T_PER_W)], ib_v)
    pltpu.sync_copy(x_hbm.at[pl.ds(base, T_PER_W)], rows_v)
    da = pltpu.async_copy(rows_v, out_hbm.at[ia_v], sema)
    db = pltpu.async_copy(rows_v, out_hbm.at[ib_v], semb)
    da.wait()
    db.wait()


def _sc_dispatch(x2, pos_a, pos_b):
    mesh = plsc.VectorSubcoreMesh(core_axis_name="c", subcore_axis_name="s")
    return pl.kernel(
        _sc_dispatch_body,
        out_type=jax.ShapeDtypeStruct((P_MAX, D_MODEL), jnp.float32),
        mesh=mesh,
        scratch_types=[
            pltpu.VMEM((T_PER_W,), jnp.int32),
            pltpu.VMEM((T_PER_W,), jnp.int32),
            pltpu.VMEM((T_PER_W, D_MODEL), jnp.float32),
            pltpu.SemaphoreType.DMA,
            pltpu.SemaphoreType.DMA,
        ],
    )(x2, pos_a, pos_b)


def _ffn_kernel(ce_ref, nreal_ref, xs_ref, w1_ref, b1_ref, w2_ref,
                b2_ref, y_ref):
    c = pl.program_id(0)

    @pl.when(c < nreal_ref[0, 0])
    def _():
        xg = xs_ref[...]
        h = lax.dot_general(xg, w1_ref[0], (((1,), (0,)), ((), ())),
                            preferred_element_type=jnp.float32)
        h = jnp.maximum(h + b1_ref[0], 0.0)
        y = lax.dot_general(h, w2_ref[0], (((1,), (0,)), ((), ())),
                            preferred_element_type=jnp.float32)
        y_ref[...] = y + b2_ref[0]


def _ffn(chunk_expert, n_real, x_rows, W1, b1, W2, b2, E, C, F):
    return pl.pallas_call(
        _ffn_kernel,
        grid_spec=pltpu.PrefetchScalarGridSpec(
            num_scalar_prefetch=2,
            grid=(N_CHUNKS,),
            in_specs=[
                pl.BlockSpec((CHUNK, C), lambda c, ce, nr: (c, 0)),
                pl.BlockSpec((1, C, F), lambda c, ce, nr: (ce[0, c], 0, 0)),
                pl.BlockSpec((1, 1, F), lambda c, ce, nr: (ce[0, c], 0, 0)),
                pl.BlockSpec((1, F, C), lambda c, ce, nr: (ce[0, c], 0, 0)),
                pl.BlockSpec((1, 1, C), lambda c, ce, nr: (ce[0, c], 0, 0)),
            ],
            out_specs=pl.BlockSpec((CHUNK, C), lambda c, ce, nr: (c, 0)),
        ),
        out_shape=jax.ShapeDtypeStruct((P_MAX, C), jnp.float32),
        compiler_params=pltpu.CompilerParams(
            dimension_semantics=("arbitrary",),
        ),
    )(chunk_expert, n_real, x_rows, W1,
      b1.reshape(E, 1, F), W2, b2.reshape(E, 1, C))


def _sc_combine_body(y_hbm, p0_hbm, p1_hbm, w0_hbm, w1_hbm, out_hbm,
                     i0_v, i1_v, w0_v, w1_v, r0_v, r1_v, sem0, sem1):
    wid = lax.axis_index("s") * SC_NC + lax.axis_index("c")
    base = wid * T_PER_W
    pltpu.sync_copy(p0_hbm.at[pl.ds(base, T_PER_W)], i0_v)
    pltpu.sync_copy(p1_hbm.at[pl.ds(base, T_PER_W)], i1_v)
    pltpu.sync_copy(w0_hbm.at[pl.ds(base, T_PER_W)], w0_v)
    pltpu.sync_copy(w1_hbm.at[pl.ds(base, T_PER_W)], w1_v)
    d0 = pltpu.async_copy(y_hbm.at[i0_v], r0_v, sem0)
    d1 = pltpu.async_copy(y_hbm.at[i1_v], r1_v, sem1)
    d0.wait()
    d1.wait()
    ncol = D_MODEL // 16

    def row_body(r, _):
        w0 = w0_v[r, :]
        w1 = w1_v[r, :]
        for cb in range(ncol):
            sl = pl.ds(cb * 16, 16)
            r0_v[r, sl] = w0 * r0_v[r, sl] + w1 * r1_v[r, sl]
        return 0

    lax.fori_loop(0, T_PER_W, row_body, 0)
    pltpu.sync_copy(r0_v, out_hbm.at[pl.ds(base, T_PER_W)])


def _sc_combine(y_rows, pos0, pos1, w0, w1):
    mesh = plsc.VectorSubcoreMesh(core_axis_name="c", subcore_axis_name="s")
    return pl.kernel(
        _sc_combine_body,
        out_type=jax.ShapeDtypeStruct((N_TOKENS, D_MODEL), jnp.float32),
        mesh=mesh,
        scratch_types=[
            pltpu.VMEM((T_PER_W,), jnp.int32),
            pltpu.VMEM((T_PER_W,), jnp.int32),
            pltpu.VMEM((T_PER_W, 16), jnp.float32),
            pltpu.VMEM((T_PER_W, 16), jnp.float32),
            pltpu.VMEM((T_PER_W, D_MODEL), jnp.float32),
            pltpu.VMEM((T_PER_W, D_MODEL), jnp.float32),
            pltpu.SemaphoreType.DMA,
            pltpu.SemaphoreType.DMA,
        ],
    )(y_rows, pos0, pos1, w0, w1)


@jax.jit
def kernel(x, Wr, W1, b1, W2, b2):
    B, N, C = x.shape
    E, _, F = W1.shape
    x2 = x.reshape(N, C)

    wa, wb, pos_a, pos_b, chunk_expert, n_real = _router(x2, Wr, N, E)
    pos_a = pos_a.reshape(N)
    pos_b = pos_b.reshape(N)

    x_rows = _sc_dispatch(x2, pos_a, pos_b)
    y_rows = _ffn(chunk_expert, n_real, x_rows, W1, b1, W2, b2, E, C, F)
    out = _sc_combine(y_rows, pos_a, pos_b, wa, wb)
    return out.reshape(B, N, C)


# speedup vs baseline: 2.3896x; 1.0535x over previous
import functools

import jax
import jax.numpy as jnp
from jax import lax
from jax.experimental import pallas as pl
from jax.experimental.pallas import tpu as pltpu
from jax.experimental.pallas import tpu_sc as plsc

N_TOKENS = 2048
D_MODEL = 768
D_FF = 1024
N_EXPERTS = 16
TOP_K = 2

CHUNK = 256
N_FLAT = N_TOKENS * TOP_K
P_MAX = 8192
N_CHUNKS = P_MAX // CHUNK

SC_NC = 2
SC_NS = 16
SC_NW = SC_NC * SC_NS
T_PER_W = N_TOKENS // SC_NW


def _cumsum0(v):
    n = v.shape[0]
    sh = 1
    while sh < n:
        shifted = jnp.concatenate(
            [jnp.zeros((sh, v.shape[1]), v.dtype), v[:-sh]], axis=0)
        v = v + shifted
        sh *= 2
    return v


def _router_kernel(x_ref, wr_ref, wa_ref, wb_ref, pa_ref, pb_ref,
                   ce_ref, nr_ref):
    x = x_ref[...]
    wr = wr_ref[...]
    logits = lax.dot_general(x, wr, (((1,), (1,)), ((), ())),
                             preferred_element_type=jnp.float32)
    m = jnp.max(logits, axis=-1, keepdims=True)
    ex = jnp.exp(logits - m)
    p = ex / jnp.sum(ex, axis=-1, keepdims=True)
    ids = lax.broadcasted_iota(jnp.int32, p.shape, 1)
    a1 = jnp.argmax(p, axis=-1)
    oh1 = ids == a1[:, None]
    w1 = jnp.sum(jnp.where(oh1, p, 0.0), axis=-1)
    a2 = jnp.argmax(jnp.where(oh1, -1.0, p), axis=-1)
    oh2 = ids == a2[:, None]
    w2 = jnp.sum(jnp.where(oh2, p, 0.0), axis=-1)
    s = w1 + w2 + 1e-9
    n, e = p.shape
    wa_ref[...] = jnp.broadcast_to((w1 / s)[:, None], (n, 16))
    wb_ref[...] = jnp.broadcast_to((w2 / s)[:, None], (n, 16))

    cnt = oh1.astype(jnp.int32) + oh2.astype(jnp.int32)
    cinc = _cumsum0(cnt)
    cexc = cinc - cnt
    sizes = cinc[n - 1:n, :].astype(jnp.float32)
    chunk_f = jnp.float32(CHUNK)
    psz = jnp.floor((sizes + (chunk_f - 1.0)) / chunk_f) * chunk_f
    rr = lax.broadcasted_iota(jnp.int32, (e, e), 0)
    cc = lax.broadcasted_iota(jnp.int32, (e, e), 1)
    upper = (rr <= cc).astype(jnp.float32)
    pends = lax.dot_general(psz, upper, (((1,), (0,)), ((), ())),
                            preferred_element_type=jnp.float32)
    poff = pends - psz
    rank_a = jnp.sum(jnp.where(oh1, cexc, 0), axis=-1)
    rank_b = jnp.sum(jnp.where(oh2, cexc, 0), axis=-1)
    poff_b = jnp.broadcast_to(poff, (n, e))
    base_a = jnp.sum(jnp.where(oh1, poff_b, 0.0), axis=-1)
    base_b = jnp.sum(jnp.where(oh2, poff_b, 0.0), axis=-1)
    pa_ref[...] = (base_a.astype(jnp.int32) + rank_a)[:, None]
    pb_ref[...] = (base_b.astype(jnp.int32) + rank_b)[:, None]

    cends = (pends / chunk_f).astype(jnp.int32)
    nr_ref[...] = cends[:, e - 1:e]
    ce_cols = lax.broadcasted_iota(jnp.int32, (e, N_CHUNKS), 1)
    cends_t = jnp.broadcast_to(cends.reshape(e, 1), (e, N_CHUNKS))
    ce = jnp.sum((cends_t <= ce_cols).astype(jnp.int32), axis=0,
                 keepdims=True)
    ce_ref[...] = jnp.minimum(ce, e - 1)


def _router(x2, Wr, N, E):
    return pl.pallas_call(
        _router_kernel,
        out_shape=(
            jax.ShapeDtypeStruct((N, 16), jnp.float32),
            jax.ShapeDtypeStruct((N, 16), jnp.float32),
            jax.ShapeDtypeStruct((N, 1), jnp.int32),
            jax.ShapeDtypeStruct((N, 1), jnp.int32),
            jax.ShapeDtypeStruct((1, N_CHUNKS), jnp.int32),
            jax.ShapeDtypeStruct((1, 1), jnp.int32),
        ),
    )(x2, Wr)


def _sc_dispatch_body(x_hbm, pa_hbm, pb_hbm, out_hbm, ia_v, ib_v, rows_v,
                      sema, semb):
    wid = lax.axis_index("s") * SC_NC + lax.axis_index("c")
    base = wid * T_PER_W
    pltpu.sync_copy(pa_hbm.at[pl.ds(base, T_PER_W)], ia_v)
    pltpu.sync_copy(pb_hbm.at[pl.ds(base, T_PER_W)], ib_v)
    pltpu.sync_copy(x_hbm.at[pl.ds(base, T_PER_W)], rows_v)
    da = pltpu.async_copy(rows_v, out_hbm.at[ia_v], sema)
    db = pltpu.async_copy(rows_v, out_hbm.at[ib_v], semb)
    da.wait()
    db.wait()


def _sc_dispatch(x2, pos_a, pos_b):
    mesh = plsc.VectorSubcoreMesh(core_axis_name="c", subcore_axis_name="s")
    return pl.kernel(
        _sc_dispatch_body,
        out_type=jax.ShapeDtypeStruct((P_MAX, D_MODEL), jnp.float32),
        mesh=mesh,
        scratch_types=[
            pltpu.VMEM((T_PER_W,), jnp.int32),
            pltpu.VMEM((T_PER_W,), jnp.int32),
            pltpu.VMEM((T_PER_W, D_MODEL), jnp.float32),
            pltpu.SemaphoreType.DMA,
            pltpu.SemaphoreType.DMA,
        ],
    )(x2, pos_a, pos_b)


def _ffn_kernel(ce_ref, nreal_ref, xs_ref, w1_ref, b1_ref, w2_ref,
                b2_ref, y_ref):
    c = pl.program_id(0)

    @pl.when(c < nreal_ref[0, 0])
    def _():
        xg = xs_ref[...]
        h = lax.dot_general(xg, w1_ref[0], (((1,), (0,)), ((), ())),
                            preferred_element_type=jnp.float32)
        h = jnp.maximum(h + b1_ref[0], 0.0)
        y = lax.dot_general(h, w2_ref[0], (((1,), (0,)), ((), ())),
                            preferred_element_type=jnp.float32)
        y_ref[...] = y + b2_ref[0]


def _ffn(chunk_expert, n_real, x_rows, W1, b1, W2, b2, E, C, F):
    return pl.pallas_call(
        _ffn_kernel,
        grid_spec=pltpu.PrefetchScalarGridSpec(
            num_scalar_prefetch=2,
            grid=(N_CHUNKS,),
            in_specs=[
                pl.BlockSpec(
                    (CHUNK, C),
                    lambda c, ce, nr: (jnp.minimum(c, nr[0, 0] - 1), 0)),
                pl.BlockSpec(
                    (1, C, F),
                    lambda c, ce, nr:
                    (ce[0, jnp.minimum(c, nr[0, 0] - 1)], 0, 0)),
                pl.BlockSpec(
                    (1, 1, F),
                    lambda c, ce, nr:
                    (ce[0, jnp.minimum(c, nr[0, 0] - 1)], 0, 0)),
                pl.BlockSpec(
                    (1, F, C),
                    lambda c, ce, nr:
                    (ce[0, jnp.minimum(c, nr[0, 0] - 1)], 0, 0)),
                pl.BlockSpec(
                    (1, 1, C),
                    lambda c, ce, nr:
                    (ce[0, jnp.minimum(c, nr[0, 0] - 1)], 0, 0)),
            ],
            out_specs=pl.BlockSpec(
                (CHUNK, C),
                lambda c, ce, nr: (jnp.minimum(c, nr[0, 0] - 1), 0)),
        ),
        out_shape=jax.ShapeDtypeStruct((P_MAX, C), jnp.float32),
        compiler_params=pltpu.CompilerParams(
            dimension_semantics=("arbitrary",),
        ),
    )(chunk_expert, n_real, x_rows, W1,
      b1.reshape(E, 1, F), W2, b2.reshape(E, 1, C))


def _sc_combine_body(y_hbm, p0_hbm, p1_hbm, w0_hbm, w1_hbm, out_hbm,
                     i0_v, i1_v, w0_v, w1_v, r0_v, r1_v, sem0, sem1):
    wid = lax.axis_index("s") * SC_NC + lax.axis_index("c")
    base = wid * T_PER_W
    pltpu.sync_copy(p0_hbm.at[pl.ds(base, T_PER_W)], i0_v)
    pltpu.sync_copy(p1_hbm.at[pl.ds(base, T_PER_W)], i1_v)
    pltpu.sync_copy(w0_hbm.at[pl.ds(base, T_PER_W)], w0_v)
    pltpu.sync_copy(w1_hbm.at[pl.ds(base, T_PER_W)], w1_v)
    d0 = pltpu.async_copy(y_hbm.at[i0_v], r0_v, sem0)
    d1 = pltpu.async_copy(y_hbm.at[i1_v], r1_v, sem1)
    d0.wait()
    d1.wait()
    ncol = D_MODEL // 16

    def row_body(r, _):
        w0 = w0_v[r, :]
        w1 = w1_v[r, :]
        for cb in range(ncol):
            sl = pl.ds(cb * 16, 16)
            r0_v[r, sl] = w0 * r0_v[r, sl] + w1 * r1_v[r, sl]
        return 0

    lax.fori_loop(0, T_PER_W, row_body, 0)
    pltpu.sync_copy(r0_v, out_hbm.at[pl.ds(base, T_PER_W)])


def _sc_combine(y_rows, pos0, pos1, w0, w1):
    mesh = plsc.VectorSubcoreMesh(core_axis_name="c", subcore_axis_name="s")
    return pl.kernel(
        _sc_combine_body,
        out_type=jax.ShapeDtypeStruct((N_TOKENS, D_MODEL), jnp.float32),
        mesh=mesh,
        scratch_types=[
            pltpu.VMEM((T_PER_W,), jnp.int32),
            pltpu.VMEM((T_PER_W,), jnp.int32),
            pltpu.VMEM((T_PER_W, 16), jnp.float32),
            pltpu.VMEM((T_PER_W, 16), jnp.float32),
            pltpu.VMEM((T_PER_W, D_MODEL), jnp.float32),
            pltpu.VMEM((T_PER_W, D_MODEL), jnp.float32),
            pltpu.SemaphoreType.DMA,
            pltpu.SemaphoreType.DMA,
        ],
    )(y_rows, pos0, pos1, w0, w1)


@jax.jit
def kernel(x, Wr, W1, b1, W2, b2):
    B, N, C = x.shape
    E, _, F = W1.shape
    x2 = x.reshape(N, C)

    wa, wb, pos_a, pos_b, chunk_expert, n_real = _router(x2, Wr, N, E)
    pos_a = pos_a.reshape(N)
    pos_b = pos_b.reshape(N)

    x_rows = _sc_dispatch(x2, pos_a, pos_b)
    y_rows = _ffn(chunk_expert, n_real, x_rows, W1, b1, W2, b2, E, C, F)
    out = _sc_combine(y_rows, pos_a, pos_b, wa, wb)
    return out.reshape(B, N, C)


# speedup vs baseline: 2.4148x; 1.0106x over previous
import functools

import jax
import jax.numpy as jnp
from jax import lax
from jax.experimental import pallas as pl
from jax.experimental.pallas import tpu as pltpu
from jax.experimental.pallas import tpu_sc as plsc

N_TOKENS = 2048
D_MODEL = 768
D_FF = 1024
N_EXPERTS = 16
TOP_K = 2

CHUNK = 256
N_FLAT = N_TOKENS * TOP_K
P_MAX = 8192
N_CHUNKS = P_MAX // CHUNK

SC_NC = 2
SC_NS = 16
SC_NW = SC_NC * SC_NS
T_PER_W = N_TOKENS // SC_NW


def _cumsum0(v):
    n = v.shape[0]
    sh = 1
    while sh < n:
        shifted = jnp.concatenate(
            [jnp.zeros((sh, v.shape[1]), v.dtype), v[:-sh]], axis=0)
        v = v + shifted
        sh *= 2
    return v


def _router_kernel(x_ref, wr_ref, wa_ref, wb_ref, pa_ref, pb_ref,
                   ce_ref, nr_ref):
    x = x_ref[...]
    wr = wr_ref[...]
    logits = lax.dot_general(x, wr, (((1,), (1,)), ((), ())),
                             preferred_element_type=jnp.float32)
    m = jnp.max(logits, axis=-1, keepdims=True)
    ex = jnp.exp(logits - m)
    p = ex / jnp.sum(ex, axis=-1, keepdims=True)
    ids = lax.broadcasted_iota(jnp.int32, p.shape, 1)
    a1 = jnp.argmax(p, axis=-1)
    oh1 = ids == a1[:, None]
    w1 = jnp.sum(jnp.where(oh1, p, 0.0), axis=-1)
    a2 = jnp.argmax(jnp.where(oh1, -1.0, p), axis=-1)
    oh2 = ids == a2[:, None]
    w2 = jnp.sum(jnp.where(oh2, p, 0.0), axis=-1)
    s = w1 + w2 + 1e-9
    n, e = p.shape
    wa_ref[...] = jnp.broadcast_to((w1 / s)[:, None], (n, 16))
    wb_ref[...] = jnp.broadcast_to((w2 / s)[:, None], (n, 16))

    cnt = oh1.astype(jnp.int32) + oh2.astype(jnp.int32)
    cinc = _cumsum0(cnt)
    cexc = cinc - cnt
    sizes = cinc[n - 1:n, :].astype(jnp.float32)
    chunk_f = jnp.float32(CHUNK)
    psz = jnp.floor((sizes + (chunk_f - 1.0)) / chunk_f) * chunk_f
    rr = lax.broadcasted_iota(jnp.int32, (e, e), 0)
    cc = lax.broadcasted_iota(jnp.int32, (e, e), 1)
    upper = (rr <= cc).astype(jnp.float32)
    pends = lax.dot_general(psz, upper, (((1,), (0,)), ((), ())),
                            preferred_element_type=jnp.float32)
    poff = pends - psz
    rank_a = jnp.sum(jnp.where(oh1, cexc, 0), axis=-1)
    rank_b = jnp.sum(jnp.where(oh2, cexc, 0), axis=-1)
    poff_b = jnp.broadcast_to(poff, (n, e))
    base_a = jnp.sum(jnp.where(oh1, poff_b, 0.0), axis=-1)
    base_b = jnp.sum(jnp.where(oh2, poff_b, 0.0), axis=-1)
    pa_ref[...] = (base_a.astype(jnp.int32) + rank_a)[:, None]
    pb_ref[...] = (base_b.astype(jnp.int32) + rank_b)[:, None]

    cends = (pends / chunk_f).astype(jnp.int32)
    nr_ref[...] = cends[:, e - 1:e]
    ce_cols = lax.broadcasted_iota(jnp.int32, (e, N_CHUNKS), 1)
    cends_t = jnp.broadcast_to(cends.reshape(e, 1), (e, N_CHUNKS))
    ce = jnp.sum((cends_t <= ce_cols).astype(jnp.int32), axis=0,
                 keepdims=True)
    ce_ref[...] = jnp.minimum(ce, e - 1)


def _router(x2, Wr, N, E):
    return pl.pallas_call(
        _router_kernel,
        out_shape=(
            jax.ShapeDtypeStruct((N, 16), jnp.float32),
            jax.ShapeDtypeStruct((N, 16), jnp.float32),
            jax.ShapeDtypeStruct((N, 1), jnp.int32),
            jax.ShapeDtypeStruct((N, 1), jnp.int32),
            jax.ShapeDtypeStruct((1, N_CHUNKS), jnp.int32),
            jax.ShapeDtypeStruct((1, 1), jnp.int32),
        ),
    )(x2, Wr)


def _sc_dispatch_body(x_hbm, pa_hbm, pb_hbm, out_hbm, ia0_v, ia1_v, ib0_v,
                      ib1_v, rows0_v, rows1_v, semr, sema, semb):
    wid = lax.axis_index("s") * SC_NC + lax.axis_index("c")
    base = wid * T_PER_W
    half = T_PER_W // 2
    pltpu.sync_copy(pa_hbm.at[pl.ds(base, half)], ia0_v)
    pltpu.sync_copy(pa_hbm.at[pl.ds(base + half, half)], ia1_v)
    pltpu.sync_copy(pb_hbm.at[pl.ds(base, half)], ib0_v)
    pltpu.sync_copy(pb_hbm.at[pl.ds(base + half, half)], ib1_v)
    dr0 = pltpu.async_copy(x_hbm.at[pl.ds(base, half)], rows0_v, semr)
    dr1 = pltpu.async_copy(x_hbm.at[pl.ds(base + half, half)], rows1_v, semr)
    dr0.wait()
    da0 = pltpu.async_copy(rows0_v, out_hbm.at[ia0_v], sema)
    db0 = pltpu.async_copy(rows0_v, out_hbm.at[ib0_v], semb)
    dr1.wait()
    da1 = pltpu.async_copy(rows1_v, out_hbm.at[ia1_v], sema)
    db1 = pltpu.async_copy(rows1_v, out_hbm.at[ib1_v], semb)
    da0.wait()
    db0.wait()
    da1.wait()
    db1.wait()


def _sc_dispatch(x2, pos_a, pos_b):
    mesh = plsc.VectorSubcoreMesh(core_axis_name="c", subcore_axis_name="s")
    half = T_PER_W // 2
    return pl.kernel(
        _sc_dispatch_body,
        out_type=jax.ShapeDtypeStruct((P_MAX, D_MODEL), jnp.float32),
        mesh=mesh,
        scratch_types=[
            pltpu.VMEM((half,), jnp.int32),
            pltpu.VMEM((half,), jnp.int32),
            pltpu.VMEM((half,), jnp.int32),
            pltpu.VMEM((half,), jnp.int32),
            pltpu.VMEM((half, D_MODEL), jnp.float32),
            pltpu.VMEM((half, D_MODEL), jnp.float32),
            pltpu.SemaphoreType.DMA,
            pltpu.SemaphoreType.DMA,
            pltpu.SemaphoreType.DMA,
        ],
    )(x2, pos_a, pos_b)


def _ffn_kernel(ce_ref, nreal_ref, xs_ref, w1_ref, b1_ref, w2_ref,
                b2_ref, y_ref):
    c = pl.program_id(0)

    @pl.when(c < nreal_ref[0, 0])
    def _():
        xg = xs_ref[...]
        h = lax.dot_general(xg, w1_ref[0], (((1,), (0,)), ((), ())),
                            preferred_element_type=jnp.float32)
        h = jnp.maximum(h + b1_ref[0], 0.0)
        y = lax.dot_general(h, w2_ref[0], (((1,), (0,)), ((), ())),
                            preferred_element_type=jnp.float32)
        y_ref[...] = y + b2_ref[0]


def _ffn(chunk_expert, n_real, x_rows, W1, b1, W2, b2, E, C, F):
    return pl.pallas_call(
        _ffn_kernel,
        grid_spec=pltpu.PrefetchScalarGridSpec(
            num_scalar_prefetch=2,
            grid=(N_CHUNKS,),
            in_specs=[
                pl.BlockSpec(
                    (CHUNK, C),
                    lambda c, ce, nr: (jnp.minimum(c, nr[0, 0] - 1), 0)),
                pl.BlockSpec(
                    (1, C, F),
                    lambda c, ce, nr:
                    (ce[0, jnp.minimum(c, nr[0, 0] - 1)], 0, 0)),
                pl.BlockSpec(
                    (1, 1, F),
                    lambda c, ce, nr:
                    (ce[0, jnp.minimum(c, nr[0, 0] - 1)], 0, 0)),
                pl.BlockSpec(
                    (1, F, C),
                    lambda c, ce, nr:
                    (ce[0, jnp.minimum(c, nr[0, 0] - 1)], 0, 0)),
                pl.BlockSpec(
                    (1, 1, C),
                    lambda c, ce, nr:
                    (ce[0, jnp.minimum(c, nr[0, 0] - 1)], 0, 0)),
            ],
            out_specs=pl.BlockSpec(
                (CHUNK, C),
                lambda c, ce, nr: (jnp.minimum(c, nr[0, 0] - 1), 0)),
        ),
        out_shape=jax.ShapeDtypeStruct((P_MAX, C), jnp.float32),
        compiler_params=pltpu.CompilerParams(
            dimension_semantics=("arbitrary",),
        ),
    )(chunk_expert, n_real, x_rows, W1,
      b1.reshape(E, 1, F), W2, b2.reshape(E, 1, C))


def _sc_combine_body(y_hbm, p0_hbm, p1_hbm, w0_hbm, w1_hbm, out_hbm,
                     i00_v, i01_v, i10_v, i11_v, w0_v, w1_v,
                     r00_v, r01_v, r10_v, r11_v, sem0, sem1, semw):
    wid = lax.axis_index("s") * SC_NC + lax.axis_index("c")
    base = wid * T_PER_W
    half = T_PER_W // 2
    pltpu.sync_copy(p0_hbm.at[pl.ds(base, half)], i00_v)
    pltpu.sync_copy(p0_hbm.at[pl.ds(base + half, half)], i01_v)
    pltpu.sync_copy(p1_hbm.at[pl.ds(base, half)], i10_v)
    pltpu.sync_copy(p1_hbm.at[pl.ds(base + half, half)], i11_v)
    d00 = pltpu.async_copy(y_hbm.at[i00_v], r00_v, sem0)
    d10 = pltpu.async_copy(y_hbm.at[i10_v], r10_v, sem0)
    d01 = pltpu.async_copy(y_hbm.at[i01_v], r01_v, sem1)
    d11 = pltpu.async_copy(y_hbm.at[i11_v], r11_v, sem1)
    pltpu.sync_copy(w0_hbm.at[pl.ds(base, T_PER_W)], w0_v)
    pltpu.sync_copy(w1_hbm.at[pl.ds(base, T_PER_W)], w1_v)
    ncol = D_MODEL // 16

    def make_row_body(ra_v, rb_v, woff):
        def row_body(r, _):
            w0 = w0_v[r + woff, :]
            w1 = w1_v[r + woff, :]
            for cb in range(ncol):
                sl = pl.ds(cb * 16, 16)
                ra_v[r, sl] = w0 * ra_v[r, sl] + w1 * rb_v[r, sl]
            return 0
        return row_body

    d00.wait()
    d10.wait()
    lax.fori_loop(0, half, make_row_body(r00_v, r10_v, 0), 0)
    dw0 = pltpu.async_copy(r00_v, out_hbm.at[pl.ds(base, half)], semw)
    d01.wait()
    d11.wait()
    lax.fori_loop(0, half, make_row_body(r01_v, r11_v, half), 0)
    dw1 = pltpu.async_copy(r01_v, out_hbm.at[pl.ds(base + half, half)], semw)
    dw0.wait()
    dw1.wait()


def _sc_combine(y_rows, pos0, pos1, w0, w1):
    mesh = plsc.VectorSubcoreMesh(core_axis_name="c", subcore_axis_name="s")
    half = T_PER_W // 2
    return pl.kernel(
        _sc_combine_body,
        out_type=jax.ShapeDtypeStruct((N_TOKENS, D_MODEL), jnp.float32),
        mesh=mesh,
        scratch_types=[
            pltpu.VMEM((half,), jnp.int32),
            pltpu.VMEM((half,), jnp.int32),
            pltpu.VMEM((half,), jnp.int32),
            pltpu.VMEM((half,), jnp.int32),
            pltpu.VMEM((T_PER_W, 16), jnp.float32),
            pltpu.VMEM((T_PER_W, 16), jnp.float32),
            pltpu.VMEM((half, D_MODEL), jnp.float32),
            pltpu.VMEM((half, D_MODEL), jnp.float32),
            pltpu.VMEM((half, D_MODEL), jnp.float32),
            pltpu.VMEM((half, D_MODEL), jnp.float32),
            pltpu.SemaphoreType.DMA,
            pltpu.SemaphoreType.DMA,
            pltpu.SemaphoreType.DMA,
        ],
    )(y_rows, pos0, pos1, w0, w1)


@jax.jit
def kernel(x, Wr, W1, b1, W2, b2):
    B, N, C = x.shape
    E, _, F = W1.shape
    x2 = x.reshape(N, C)

    wa, wb, pos_a, pos_b, chunk_expert, n_real = _router(x2, Wr, N, E)
    pos_a = pos_a.reshape(N)
    pos_b = pos_b.reshape(N)

    x_rows = _sc_dispatch(x2, pos_a, pos_b)
    y_rows = _ffn(chunk_expert, n_real, x_rows, W1, b1, W2, b2, E, C, F)
    out = _sc_combine(y_rows, pos_a, pos_b, wa, wb)
    return out.reshape(B, N, C)


# speedup vs baseline: 2.4672x; 1.0217x over previous
import functools

import jax
import jax.numpy as jnp
from jax import lax
from jax.experimental import pallas as pl
from jax.experimental.pallas import tpu as pltpu
from jax.experimental.pallas import tpu_sc as plsc

N_TOKENS = 2048
D_MODEL = 768
D_FF = 1024
N_EXPERTS = 16
TOP_K = 2

CHUNK = 256
N_FLAT = N_TOKENS * TOP_K
P_MAX = 8192
N_CHUNKS = P_MAX // CHUNK

SC_NC = 2
SC_NS = 16
SC_NW = SC_NC * SC_NS
T_PER_W = N_TOKENS // SC_NW


def _cumsum0(v):
    n = v.shape[0]
    sh = 1
    while sh < n:
        shifted = jnp.concatenate(
            [jnp.zeros((sh, v.shape[1]), v.dtype), v[:-sh]], axis=0)
        v = v + shifted
        sh *= 2
    return v


def _router_kernel(x_ref, wr_ref, wa_ref, wb_ref, pa_ref, pb_ref,
                   ce_ref, nr_ref, xb_ref):
    x = x_ref[...]
    xi16 = pltpu.bitcast(x.astype(jnp.bfloat16), jnp.int16)
    h0 = xi16[:, :D_MODEL // 2].astype(jnp.int32) & 0xFFFF
    h1 = xi16[:, D_MODEL // 2:].astype(jnp.int32)
    xb_ref[...] = (h1 << 16) | h0
    wr = wr_ref[...]
    logits = lax.dot_general(x, wr, (((1,), (1,)), ((), ())),
                             preferred_element_type=jnp.float32)
    m = jnp.max(logits, axis=-1, keepdims=True)
    ex = jnp.exp(logits - m)
    p = ex / jnp.sum(ex, axis=-1, keepdims=True)
    ids = lax.broadcasted_iota(jnp.int32, p.shape, 1)
    a1 = jnp.argmax(p, axis=-1)
    oh1 = ids == a1[:, None]
    w1 = jnp.sum(jnp.where(oh1, p, 0.0), axis=-1)
    a2 = jnp.argmax(jnp.where(oh1, -1.0, p), axis=-1)
    oh2 = ids == a2[:, None]
    w2 = jnp.sum(jnp.where(oh2, p, 0.0), axis=-1)
    s = w1 + w2 + 1e-9
    n, e = p.shape
    wa_ref[...] = jnp.broadcast_to((w1 / s)[:, None], (n, 16))
    wb_ref[...] = jnp.broadcast_to((w2 / s)[:, None], (n, 16))

    cnt = oh1.astype(jnp.int32) + oh2.astype(jnp.int32)
    cinc = _cumsum0(cnt)
    cexc = cinc - cnt
    sizes = cinc[n - 1:n, :].astype(jnp.float32)
    chunk_f = jnp.float32(CHUNK)
    psz = jnp.floor((sizes + (chunk_f - 1.0)) / chunk_f) * chunk_f
    rr = lax.broadcasted_iota(jnp.int32, (e, e), 0)
    cc = lax.broadcasted_iota(jnp.int32, (e, e), 1)
    upper = (rr <= cc).astype(jnp.float32)
    pends = lax.dot_general(psz, upper, (((1,), (0,)), ((), ())),
                            preferred_element_type=jnp.float32)
    poff = pends - psz
    rank_a = jnp.sum(jnp.where(oh1, cexc, 0), axis=-1)
    rank_b = jnp.sum(jnp.where(oh2, cexc, 0), axis=-1)
    poff_b = jnp.broadcast_to(poff, (n, e))
    base_a = jnp.sum(jnp.where(oh1, poff_b, 0.0), axis=-1)
    base_b = jnp.sum(jnp.where(oh2, poff_b, 0.0), axis=-1)
    pa_ref[...] = (base_a.astype(jnp.int32) + rank_a)[:, None]
    pb_ref[...] = (base_b.astype(jnp.int32) + rank_b)[:, None]

    cends = (pends / chunk_f).astype(jnp.int32)
    nr_ref[...] = cends[:, e - 1:e]
    ce_cols = lax.broadcasted_iota(jnp.int32, (e, N_CHUNKS), 1)
    cends_t = jnp.broadcast_to(cends.reshape(e, 1), (e, N_CHUNKS))
    ce = jnp.sum((cends_t <= ce_cols).astype(jnp.int32), axis=0,
                 keepdims=True)
    ce_ref[...] = jnp.minimum(ce, e - 1)


def _router(x2, Wr, N, E):
    return pl.pallas_call(
        _router_kernel,
        out_shape=(
            jax.ShapeDtypeStruct((N, 16), jnp.float32),
            jax.ShapeDtypeStruct((N, 16), jnp.float32),
            jax.ShapeDtypeStruct((N, 1), jnp.int32),
            jax.ShapeDtypeStruct((N, 1), jnp.int32),
            jax.ShapeDtypeStruct((1, N_CHUNKS), jnp.int32),
            jax.ShapeDtypeStruct((1, 1), jnp.int32),
            jax.ShapeDtypeStruct((N, D_MODEL // 2), jnp.int32),
        ),
    )(x2, Wr)


def _sc_dispatch_body(x_hbm, pa_hbm, pb_hbm, out_hbm, ia0_v, ia1_v, ib0_v,
                      ib1_v, rows0_v, rows1_v, semr, sema, semb):
    wid = lax.axis_index("s") * SC_NC + lax.axis_index("c")
    base = wid * T_PER_W
    half = T_PER_W // 2
    pltpu.sync_copy(pa_hbm.at[pl.ds(base, half)], ia0_v)
    pltpu.sync_copy(pa_hbm.at[pl.ds(base + half, half)], ia1_v)
    pltpu.sync_copy(pb_hbm.at[pl.ds(base, half)], ib0_v)
    pltpu.sync_copy(pb_hbm.at[pl.ds(base + half, half)], ib1_v)
    dr0 = pltpu.async_copy(x_hbm.at[pl.ds(base, half)], rows0_v, semr)
    dr1 = pltpu.async_copy(x_hbm.at[pl.ds(base + half, half)], rows1_v, semr)
    dr0.wait()
    da0 = pltpu.async_copy(rows0_v, out_hbm.at[ia0_v], sema)
    db0 = pltpu.async_copy(rows0_v, out_hbm.at[ib0_v], semb)
    dr1.wait()
    da1 = pltpu.async_copy(rows1_v, out_hbm.at[ia1_v], sema)
    db1 = pltpu.async_copy(rows1_v, out_hbm.at[ib1_v], semb)
    da0.wait()
    db0.wait()
    da1.wait()
    db1.wait()


def _sc_dispatch(x2, pos_a, pos_b):
    mesh = plsc.VectorSubcoreMesh(core_axis_name="c", subcore_axis_name="s")
    half = T_PER_W // 2
    return pl.kernel(
        _sc_dispatch_body,
        out_type=jax.ShapeDtypeStruct((P_MAX, D_MODEL // 2), jnp.int32),
        mesh=mesh,
        scratch_types=[
            pltpu.VMEM((half,), jnp.int32),
            pltpu.VMEM((half,), jnp.int32),
            pltpu.VMEM((half,), jnp.int32),
            pltpu.VMEM((half,), jnp.int32),
            pltpu.VMEM((half, D_MODEL // 2), jnp.int32),
            pltpu.VMEM((half, D_MODEL // 2), jnp.int32),
            pltpu.SemaphoreType.DMA,
            pltpu.SemaphoreType.DMA,
            pltpu.SemaphoreType.DMA,
        ],
    )(x2, pos_a, pos_b)


def _ffn_kernel(ce_ref, nreal_ref, xs_ref, w1_ref, b1_ref, w2_ref,
                b2_ref, y_ref):
    c = pl.program_id(0)

    @pl.when(c < nreal_ref[0, 0])
    def _():
        xi = xs_ref[...]
        lo = pltpu.bitcast(((xi << 16) >> 16).astype(jnp.int16),
                           jnp.bfloat16).astype(jnp.float32)
        hi = pltpu.bitcast((xi >> 16).astype(jnp.int16),
                           jnp.bfloat16).astype(jnp.float32)
        half_c = xi.shape[1]
        h = (lax.dot_general(lo, w1_ref[0][:half_c], (((1,), (0,)), ((), ())),
                             preferred_element_type=jnp.float32)
             + lax.dot_general(hi, w1_ref[0][half_c:],
                               (((1,), (0,)), ((), ())),
                               preferred_element_type=jnp.float32))
        h = jnp.maximum(h + b1_ref[0], 0.0)
        y = lax.dot_general(h, w2_ref[0], (((1,), (0,)), ((), ())),
                            preferred_element_type=jnp.float32)
        y_ref[...] = y + b2_ref[0]


def _ffn(chunk_expert, n_real, x_rows, W1, b1, W2, b2, E, C, F):
    return pl.pallas_call(
        _ffn_kernel,
        grid_spec=pltpu.PrefetchScalarGridSpec(
            num_scalar_prefetch=2,
            grid=(N_CHUNKS,),
            in_specs=[
                pl.BlockSpec(
                    (CHUNK, C // 2),
                    lambda c, ce, nr: (jnp.minimum(c, nr[0, 0] - 1), 0)),
                pl.BlockSpec(
                    (1, C, F),
                    lambda c, ce, nr:
                    (ce[0, jnp.minimum(c, nr[0, 0] - 1)], 0, 0)),
                pl.BlockSpec(
                    (1, 1, F),
                    lambda c, ce, nr:
                    (ce[0, jnp.minimum(c, nr[0, 0] - 1)], 0, 0)),
                pl.BlockSpec(
                    (1, F, C),
                    lambda c, ce, nr:
                    (ce[0, jnp.minimum(c, nr[0, 0] - 1)], 0, 0)),
                pl.BlockSpec(
                    (1, 1, C),
                    lambda c, ce, nr:
                    (ce[0, jnp.minimum(c, nr[0, 0] - 1)], 0, 0)),
            ],
            out_specs=pl.BlockSpec(
                (CHUNK, C),
                lambda c, ce, nr: (jnp.minimum(c, nr[0, 0] - 1), 0)),
        ),
        out_shape=jax.ShapeDtypeStruct((P_MAX, C), jnp.float32),
        compiler_params=pltpu.CompilerParams(
            dimension_semantics=("arbitrary",),
        ),
    )(chunk_expert, n_real, x_rows, W1,
      b1.reshape(E, 1, F), W2, b2.reshape(E, 1, C))


def _sc_combine_body(y_hbm, p0_hbm, p1_hbm, w0_hbm, w1_hbm, out_hbm,
                     i00_v, i01_v, i10_v, i11_v, w0_v, w1_v,
                     r00_v, r01_v, r10_v, r11_v, sem0, sem1, semw):
    wid = lax.axis_index("s") * SC_NC + lax.axis_index("c")
    base = wid * T_PER_W
    half = T_PER_W // 2
    pltpu.sync_copy(p0_hbm.at[pl.ds(base, half)], i00_v)
    pltpu.sync_copy(p0_hbm.at[pl.ds(base + half, half)], i01_v)
    pltpu.sync_copy(p1_hbm.at[pl.ds(base, half)], i10_v)
    pltpu.sync_copy(p1_hbm.at[pl.ds(base + half, half)], i11_v)
    d00 = pltpu.async_copy(y_hbm.at[i00_v], r00_v, sem0)
    d10 = pltpu.async_copy(y_hbm.at[i10_v], r10_v, sem0)
    d01 = pltpu.async_copy(y_hbm.at[i01_v], r01_v, sem1)
    d11 = pltpu.async_copy(y_hbm.at[i11_v], r11_v, sem1)
    pltpu.sync_copy(w0_hbm.at[pl.ds(base, T_PER_W)], w0_v)
    pltpu.sync_copy(w1_hbm.at[pl.ds(base, T_PER_W)], w1_v)
    ncol = D_MODEL // 16

    def make_row_body(ra_v, rb_v, woff):
        def row_body(r, _):
            w0 = w0_v[r + woff, :]
            w1 = w1_v[r + woff, :]
            for cb in range(ncol):
                sl = pl.ds(cb * 16, 16)
                ra_v[r, sl] = w0 * ra_v[r, sl] + w1 * rb_v[r, sl]
            return 0
        return row_body

    d00.wait()
    d10.wait()
    lax.fori_loop(0, half, make_row_body(r00_v, r10_v, 0), 0)
    dw0 = pltpu.async_copy(r00_v, out_hbm.at[pl.ds(base, half)], semw)
    d01.wait()
    d11.wait()
    lax.fori_loop(0, half, make_row_body(r01_v, r11_v, half), 0)
    dw1 = pltpu.async_copy(r01_v, out_hbm.at[pl.ds(base + half, half)], semw)
    dw0.wait()
    dw1.wait()


def _sc_combine(y_rows, pos0, pos1, w0, w1):
    mesh = plsc.VectorSubcoreMesh(core_axis_name="c", subcore_axis_name="s")
    half = T_PER_W // 2
    return pl.kernel(
        _sc_combine_body,
        out_type=jax.ShapeDtypeStruct((N_TOKENS, D_MODEL), jnp.float32),
        mesh=mesh,
        scratch_types=[
            pltpu.VMEM((half,), jnp.int32),
            pltpu.VMEM((half,), jnp.int32),
            pltpu.VMEM((half,), jnp.int32),
            pltpu.VMEM((half,), jnp.int32),
            pltpu.VMEM((T_PER_W, 16), jnp.float32),
            pltpu.VMEM((T_PER_W, 16), jnp.float32),
            pltpu.VMEM((half, D_MODEL), jnp.float32),
            pltpu.VMEM((half, D_MODEL), jnp.float32),
            pltpu.VMEM((half, D_MODEL), jnp.float32),
            pltpu.VMEM((half, D_MODEL), jnp.float32),
            pltpu.SemaphoreType.DMA,
            pltpu.SemaphoreType.DMA,
            pltpu.SemaphoreType.DMA,
        ],
    )(y_rows, pos0, pos1, w0, w1)


@jax.jit
def kernel(x, Wr, W1, b1, W2, b2):
    B, N, C = x.shape
    E, _, F = W1.shape
    x2 = x.reshape(N, C)

    wa, wb, pos_a, pos_b, chunk_expert, n_real, x_bf16 = _router(x2, Wr, N, E)
    pos_a = pos_a.reshape(N)
    pos_b = pos_b.reshape(N)

    x_rows = _sc_dispatch(x_bf16, pos_a, pos_b)
    y_rows = _ffn(chunk_expert, n_real, x_rows, W1, b1, W2, b2, E, C, F)
    out = _sc_combine(y_rows, pos_a, pos_b, wa, wb)
    return out.reshape(B, N, C)
